# baseline (speedup 1.0000x reference)
import functools

import jax
import jax.numpy as jnp
from jax import lax
from jax.experimental import pallas as pl
from jax.experimental.pallas import tpu as pltpu
from jax.experimental.pallas import tpu_sc as plsc

N = 10000
NP = 10016
D = 128
E = 320000
CHUNK = 64
GROUPS = 20
EPT = GROUPS * 8 * CHUNK
NC, NS = 2, 16
NW = NC * NS
EPAD = NW * EPT
SUB_ROWS = 624
LAST_ROWS = N - 15 * SUB_ROWS


def _mm_body(x_ref, w_ref, o_ref):
    o_ref[...] = lax.dot_general(
        x_ref[...], w_ref[...], (((1,), (1,)), ((), ())),
        preferred_element_type=jnp.float32)


def _matmul_xwT(x, w):
    blk = 1000
    return pl.pallas_call(
        _mm_body,
        grid=(N // blk,),
        in_specs=[pl.BlockSpec((blk, D), lambda i: (i, 0)),
                  pl.BlockSpec((D, D), lambda i: (0, 0))],
        out_specs=pl.BlockSpec((blk, D), lambda i: (i, 0)),
        out_shape=jax.ShapeDtypeStruct((NP, D), jnp.float32),
    )(x, w)


def _step_body(a_ref, h_ref, w_ref, acc_ref, ws_ref, nh_ref, acco_ref):
    a = a_ref[0] + a_ref[1]
    z = lax.dot_general(a, w_ref[...], (((1,), (1,)), ((), ())),
                        preferred_element_type=jnp.float32)
    hb = h_ref[...]
    nh = jnp.maximum(z + hb, 0.0) + hb
    nh_ref[...] = nh
    acco_ref[...] = acc_ref[...] + ws_ref[0, 0] * nh


def _step_tc(agg2, h, w_pc, acc, wstep):
    blk = 1000
    return pl.pallas_call(
        _step_body,
        grid=(N // blk,),
        in_specs=[pl.BlockSpec((2, blk, D), lambda i: (0, i, 0)),
                  pl.BlockSpec((blk, D), lambda i: (i, 0)),
                  pl.BlockSpec((D, D), lambda i: (0, 0)),
                  pl.BlockSpec((blk, D), lambda i: (i, 0)),
                  pl.BlockSpec(memory_space=pltpu.SMEM)],
        out_specs=[pl.BlockSpec((blk, D), lambda i: (i, 0)),
                   pl.BlockSpec((blk, D), lambda i: (i, 0))],
        out_shape=[jax.ShapeDtypeStruct((NP, D), jnp.float32),
                   jax.ShapeDtypeStruct((N, D), jnp.float32)],
        input_output_aliases={3: 1},
    )(agg2, h, w_pc, acc, wstep)


_mesh = plsc.VectorSubcoreMesh(core_axis_name="c", subcore_axis_name="s")


@functools.partial(
    pl.kernel,
    mesh=_mesh,
    out_type=jax.ShapeDtypeStruct((NC, NP, D), jnp.float32),
    scratch_types=[
        pltpu.VMEM((8, CHUNK), jnp.int32),
        pltpu.VMEM((8, CHUNK), jnp.int32),
        pltpu.VMEM((8, CHUNK), jnp.int32),
        pltpu.VMEM((8, CHUNK), jnp.int32),
        pltpu.VMEM((CHUNK, D), jnp.float32),
        pltpu.VMEM((CHUNK, D), jnp.float32),
        pltpu.VMEM((CHUNK, D), jnp.float32),
        pltpu.VMEM((CHUNK, D), jnp.float32),
        pltpu.VMEM((16, D), jnp.float32),
        pltpu.VMEM_SHARED((NP, D), jnp.float32),
        pltpu.SemaphoreType.DMA,
        pltpu.SemaphoreType.DMA,
        pltpu.SemaphoreType.DMA,
        pltpu.SemaphoreType.DMA,
    ],
)
def _sc_agg(h_hbm, p_hbm, c_hbm, out_hbm,
            ap_v, ac_v, bp_v, bc_v, hp0_v, hp1_v, hc0_v, hc1_v, z_v, agg_sh,
            sem_a, sem_b, gsem0, gsem1):
    c = lax.axis_index("c")
    s = lax.axis_index("s")
    w = s * NC + c

    ibufs = ((ap_v, ac_v, sem_a), (bp_v, bc_v, sem_b))
    gbufs = ((hp0_v, hc0_v, gsem0), (hp1_v, hc1_v, gsem1))

    def _icopy(g, ib):
        ip_v, ic_v, sem = ibufs[ib]
        pltpu.async_copy(p_hbm.at[w, g], ip_v, sem)
        pltpu.async_copy(c_hbm.at[w, g], ic_v, sem)

    def _iwait(ib):
        ip_v, ic_v, sem = ibufs[ib]
        pltpu.make_async_copy(p_hbm.at[0, 0], ip_v, sem).wait()
        pltpu.make_async_copy(c_hbm.at[0, 0], ic_v, sem).wait()

    def _gissue(ib, k, gb):
        ip_v, ic_v, _ = ibufs[ib]
        hp_v, hc_v, sem = gbufs[gb]
        pltpu.async_copy(h_hbm.at[ip_v.at[k]], hp_v, sem)
        pltpu.async_copy(h_hbm.at[ic_v.at[k]], hc_v, sem)

    def _gwait(gb):
        hp_v, hc_v, sem = gbufs[gb]
        pltpu.make_async_copy(h_hbm.at[ap_v.at[0]], hp_v, sem).wait()
        pltpu.make_async_copy(h_hbm.at[ap_v.at[0]], hc_v, sem).wait()

    def _process(ib, k, gb):
        ip_v, ic_v, _ = ibufs[ib]
        hp_v, hc_v, _ = gbufs[gb]
        _gwait(gb)

        def _addrow(ii, cc):
            base = ii * 4
            for q in range(4):
                for kk in range(D // 16):
                    plsc.addupdate(hp_v.at[base + q, pl.ds(kk * 16, 16)],
                                   hc_v[base + q, pl.ds(kk * 16, 16)])
            return cc

        lax.fori_loop(0, CHUNK // 4, _addrow, 0)
        pltpu.sync_copy(hp_v, agg_sh.at[ip_v.at[k]], add=True)
        pltpu.sync_copy(hp_v, agg_sh.at[ic_v.at[k]], add=True)

    zero16 = jnp.zeros((16,), jnp.float32)

    def _zb(i, carry):
        for k in range(D // 16):
            z_v[i, pl.ds(k * 16, 16)] = zero16
        return carry

    lax.fori_loop(0, 16, _zb, 0)
    nz = jnp.where(s == NS - 1, LAST_ROWS // 16, SUB_ROWS // 16)

    def _zcopy(j, carry):
        pltpu.sync_copy(z_v, agg_sh.at[pl.ds(s * SUB_ROWS + j * 16, 16)])
        return carry

    lax.fori_loop(0, nz, _zcopy, 0)
    plsc.subcore_barrier()

    _icopy(0, 0)
    _iwait(0)
    _gissue(0, 0, 0)

    def _pair(t, carry):
        g0 = t * 2
        _icopy(g0 + 1, 1)
        for k in range(8):
            if k < 7:
                _gissue(0, k + 1, (k + 1) % 2)
            else:
                _iwait(1)
                _gissue(1, 0, 0)
            _process(0, k, k % 2)
        _icopy(jnp.minimum(g0 + 2, GROUPS - 1), 0)
        for k in range(8):
            if k < 7:
                _gissue(1, k + 1, (k + 1) % 2)
            else:
                _iwait(0)
                _gissue(0, 0, 0)
            _process(1, k, k % 2)
        return carry

    lax.fori_loop(0, GROUPS // 2, _pair, 0)
    _gwait(0)

    plsc.subcore_barrier()

    @pl.when(s < NS - 1)
    def _wb_main():
        pltpu.sync_copy(agg_sh.at[pl.ds(s * SUB_ROWS, SUB_ROWS)],
                        out_hbm.at[c, pl.ds(s * SUB_ROWS, SUB_ROWS)])

    @pl.when(s == NS - 1)
    def _wb_last():
        pltpu.sync_copy(agg_sh.at[pl.ds(15 * SUB_ROWS, LAST_ROWS)],
                        out_hbm.at[c, pl.ds(15 * SUB_ROWS, LAST_ROWS)])


def kernel(node_feats, edge_index, W_s, W_pc, T):
    pad = jnp.full((EPAD - E,), N, dtype=jnp.int32)
    p4 = jnp.concatenate([edge_index[0], pad]).reshape(NW, GROUPS, 8, CHUNK)
    c4 = jnp.concatenate([edge_index[1], pad]).reshape(NW, GROUPS, 8, CHUNK)
    h = _matmul_xwT(node_feats, W_s)
    weights = jax.nn.sigmoid(T - jnp.arange(3, dtype=jnp.float32))
    acc = jnp.zeros((N, D), jnp.float32)
    for step in range(3):
        agg2 = _sc_agg(h, p4, c4)
        h, acc = _step_tc(agg2, h, W_pc, acc,
                          weights[step].reshape(1, 1))
    return acc
